# Initial kernel scaffold; baseline (speedup 1.0000x reference)
import jax
import jax.numpy as jnp
from jax.experimental import pallas as pl


def kernel(x, train_hidden, train_hidden_day, W_ih0, W_hh0, b_ih0, b_hh0, W_ih1, W_hh1, b_ih1, b_hh1, lin0_W, lin0_b, lin1_W, lin1_b, proj1_W, proj2_W, fc_W, fc_b, k_day, n_neighbor):
    raise NotImplementedError("write your pallas kernel here")


# speedup vs baseline: 2.9857x; 2.9857x over previous
import jax
import jax.numpy as jnp
from jax.experimental import pallas as pl
from jax.experimental.pallas import tpu as pltpu

D_FEAT = 6
HID = 64
SEQ = 60
N = 1024
DAYS = 500
STOCKS = 1024
KTOP = 10
ROW_BLK = 128

_F32 = jnp.float32


def _mm_t(a, b):
    return jax.lax.dot_general(a, b, (((1,), (1,)), ((), ())),
                               preferred_element_type=_F32)


def _mm(a, b):
    return jax.lax.dot_general(a, b, (((1,), (0,)), ((), ())),
                               preferred_element_type=_F32)


def _leaky(v):
    return jnp.where(v >= 0, v, 0.01 * v)


def _encoder_kernel(xbT_ref, Wih0_ref, Whh0_ref, bih0_ref, bhh0_ref,
                    Wih1_ref, Whh1_ref, bih1_ref, bhh1_ref,
                    lin0W_ref, lin0b_ref, lin1W_ref, lin1b_ref,
                    proj1W_ref, thd_ref,
                    mbo_ref, p1_ref, dayidx_ref):
    Wih0 = Wih0_ref[...]
    Whh0 = Whh0_ref[...]
    bih0 = bih0_ref[...]
    bhh0 = bhh0_ref[...]
    Wih1 = Wih1_ref[...]
    Whh1 = Whh1_ref[...]
    bih1 = bih1_ref[...]
    bhh1 = bhh1_ref[...]

    def gru_step(x_t, h, Wih, Whh, bih, bhh):
        gi = _mm_t(x_t, Wih) + bih
        gh = _mm_t(h, Whh) + bhh
        r = jax.nn.sigmoid(gi[:, :HID] + gh[:, :HID])
        z = jax.nn.sigmoid(gi[:, HID:2 * HID] + gh[:, HID:2 * HID])
        n = jnp.tanh(gi[:, 2 * HID:] + r * gh[:, 2 * HID:])
        return (1.0 - z) * n + z * h

    def step(t, carry):
        h0, h1 = carry
        x_t = xbT_ref[pl.ds(t, 1), :, :].reshape(N, D_FEAT)
        h0 = gru_step(x_t, h0, Wih0, Whh0, bih0, bhh0)
        h1 = gru_step(h0, h1, Wih1, Whh1, bih1, bhh1)
        return (h0, h1)

    h0 = jnp.zeros((N, HID), _F32)
    h1 = jnp.zeros((N, HID), _F32)
    _, out = jax.lax.fori_loop(0, SEQ, step, (h0, h1))

    mbo = _leaky(_mm_t(out, lin0W_ref[...]) + lin0b_ref[...])
    mbo = _leaky(_mm_t(mbo, lin1W_ref[...]) + lin1b_ref[...])
    mbo_ref[...] = mbo
    p1_ref[...] = _mm_t(mbo, proj1W_ref[...])

    thd = thd_ref[...]
    mbd = jnp.mean(mbo, axis=0, keepdims=True)
    num = _mm_t(mbd, thd)
    an = jnp.sqrt(jnp.sum(mbd * mbd))
    bn = jnp.sqrt(_mm_t(jnp.ones((1, HID), _F32), thd * thd))
    sim = num / (an * bn + 1e-6)

    iota = jax.lax.broadcasted_iota(jnp.int32, (1, DAYS), 1)
    lane = jax.lax.broadcasted_iota(jnp.int32, (1, 128), 1)
    vec = jnp.zeros((1, 128), jnp.int32)
    work = sim
    for i in range(KTOP):
        m = jnp.max(work)
        idx = jnp.min(jnp.where(work == m, iota, jnp.int32(2 ** 30)))
        vec = jnp.where(lane == i, idx, vec)
        work = jnp.where(iota == idx, -1e30, work)
    dayidx_ref[...] = vec


def _gather_proj_kernel(dayidx_ref, th_ref, lin0W_ref, lin0b_ref,
                        lin1W_ref, lin1b_ref, proj2W_ref, p2_ref):
    del dayidx_ref
    s = th_ref[0]
    s = _leaky(_mm_t(s, lin0W_ref[...]) + lin0b_ref[...])
    s = _leaky(_mm_t(s, lin1W_ref[...]) + lin1b_ref[...])
    p2_ref[0] = _mm_t(s, proj2W_ref[...])


def _knn_kernel(p1_ref, mbo_ref, p2_ref, fcWa_ref, fcWb_ref, fcb_ref,
                pred_ref):
    p1b = p1_ref[...]
    p2 = p2_ref[...]
    num = _mm_t(p1b, p2)
    n1 = jnp.sqrt(jnp.sum(p1b * p1b, axis=1, keepdims=True))
    n2 = jnp.sqrt(_mm_t(jnp.ones((1, HID), _F32), p2 * p2))
    cs = num / (n1 * n2 + 1e-6)

    work = cs
    thresh = None
    for i in range(KTOP):
        thresh = jnp.max(work, axis=1, keepdims=True)
        if i < KTOP - 1:
            work = jnp.where(work == thresh, -1e30, work)
    w = jnp.where(cs >= thresh, cs, 0.0)
    agg = _mm(w, p2)
    pred = (_mm_t(fcWa_ref[...], mbo_ref[...])
            + _mm_t(fcWb_ref[...], agg) + fcb_ref[...])
    pred_ref[...] = pred.reshape(1, 1, ROW_BLK)


def kernel(x, train_hidden, train_hidden_day, W_ih0, W_hh0, b_ih0, b_hh0,
           W_ih1, W_hh1, b_ih1, b_hh1, lin0_W, lin0_b, lin1_W, lin1_b,
           proj1_W, proj2_W, fc_W, fc_b, k_day, n_neighbor):
    del k_day
    xbT = x.reshape(N, D_FEAT, SEQ).transpose(2, 0, 1)

    mbo, p1, dayvec = pl.pallas_call(
        _encoder_kernel,
        out_shape=[
            jax.ShapeDtypeStruct((N, HID), _F32),
            jax.ShapeDtypeStruct((N, HID), _F32),
            jax.ShapeDtypeStruct((1, 128), jnp.int32),
        ],
    )(xbT, W_ih0, W_hh0, b_ih0, b_hh0, W_ih1, W_hh1, b_ih1, b_hh1,
      lin0_W, lin0_b, lin1_W, lin1_b, proj1_W, train_hidden_day)

    day_idx = dayvec[0, :KTOP]

    grid_spec = pltpu.PrefetchScalarGridSpec(
        num_scalar_prefetch=1,
        grid=(KTOP,),
        in_specs=[
            pl.BlockSpec((1, STOCKS, HID), lambda i, idx: (idx[i], 0, 0)),
            pl.BlockSpec((HID, HID), lambda i, idx: (0, 0)),
            pl.BlockSpec((HID,), lambda i, idx: (0,)),
            pl.BlockSpec((HID, HID), lambda i, idx: (0, 0)),
            pl.BlockSpec((HID,), lambda i, idx: (0,)),
            pl.BlockSpec((HID, HID), lambda i, idx: (0, 0)),
        ],
        out_specs=pl.BlockSpec((1, STOCKS, HID), lambda i, idx: (i, 0, 0)),
    )
    p2 = pl.pallas_call(
        _gather_proj_kernel,
        grid_spec=grid_spec,
        out_shape=jax.ShapeDtypeStruct((KTOP, STOCKS, HID), _F32),
    )(day_idx, train_hidden, lin0_W, lin0_b, lin1_W, lin1_b, proj2_W)

    p2f = p2.reshape(KTOP * STOCKS, HID)
    fcWa = fc_W[:, :HID]
    fcWb = fc_W[:, HID:] / n_neighbor
    fcb = fc_b.reshape(1, 1)

    n_blk = N // ROW_BLK
    pred = pl.pallas_call(
        _knn_kernel,
        grid=(n_blk,),
        in_specs=[
            pl.BlockSpec((ROW_BLK, HID), lambda i: (i, 0)),
            pl.BlockSpec((ROW_BLK, HID), lambda i: (i, 0)),
            pl.BlockSpec((KTOP * STOCKS, HID), lambda i: (0, 0)),
            pl.BlockSpec((1, HID), lambda i: (0, 0)),
            pl.BlockSpec((1, HID), lambda i: (0, 0)),
            pl.BlockSpec((1, 1), lambda i: (0, 0)),
        ],
        out_specs=pl.BlockSpec((1, 1, ROW_BLK), lambda i: (i, 0, 0)),
        out_shape=jax.ShapeDtypeStruct((n_blk, 1, ROW_BLK), _F32),
    )(p1, mbo, p2f, fcWa, fcWb, fcb)

    return pred.reshape(N)
